# Initial kernel scaffold; baseline (speedup 1.0000x reference)
import jax
import jax.numpy as jnp
from jax.experimental import pallas as pl


def kernel(x, edge_index, batch, W1_0, W2_0, g0, b0, W1_1, W2_1, g1, b1, W1_2, W2_2):
    raise NotImplementedError("write your pallas kernel here")


# speedup vs baseline: 4.9540x; 4.9540x over previous
import functools

import jax
import jax.numpy as jnp
from jax import lax
from jax.experimental import pallas as pl
from jax.experimental.pallas import tpu as pltpu
from jax.experimental.pallas import tpu_sc as plsc

N_NODES = 10000
N_GRAPHS = 64
D = 128
E = 320000
BN_EPS = 1e-5

NC = 2
NS = 16
NW = NC * NS
EPW = E // NW
CH = 80
NCHUNK = EPW // CH
ROWS_PER_TILE = 632
NPAD = ROWS_PER_TILE * NS

_mesh = plsc.VectorSubcoreMesh(
    core_axis_name="c", subcore_axis_name="s", num_cores=NC, num_subcores=NS
)


@functools.partial(
    pl.kernel,
    mesh=_mesh,
    out_type=jax.ShapeDtypeStruct((NC, NPAD, D), jnp.float32),
    scratch_types=[
        pltpu.VMEM((CH,), jnp.int32),
        pltpu.VMEM((CH,), jnp.int32),
        pltpu.VMEM((CH, D), jnp.float32),
        pltpu.VMEM_SHARED((NPAD, D), jnp.float32),
        pltpu.SemaphoreType.DMA,
    ],
)
def _sc_segsum(h_hbm, src_hbm, dst_hbm, zeros_hbm, out_hbm,
               src_v, dst_v, rows_v, agg_sh, sem):
    c = lax.axis_index("c")
    s = lax.axis_index("s")
    wid = s * NC + c

    row0 = pl.multiple_of(s * ROWS_PER_TILE, 8)
    pltpu.sync_copy(zeros_hbm.at[pl.ds(row0, ROWS_PER_TILE)],
                    agg_sh.at[pl.ds(row0, ROWS_PER_TILE)])
    plsc.subcore_barrier()

    base = wid * EPW

    def body(i, carry):
        off = pl.multiple_of(base + i * CH, 8)
        pltpu.sync_copy(src_hbm.at[pl.ds(off, CH)], src_v)
        pltpu.sync_copy(dst_hbm.at[pl.ds(off, CH)], dst_v)
        pltpu.async_copy(h_hbm.at[src_v], rows_v, sem).wait()
        pltpu.sync_copy(rows_v, agg_sh.at[dst_v], add=True)
        return carry

    lax.fori_loop(0, NCHUNK, body, 0)
    plsc.subcore_barrier()

    pltpu.sync_copy(agg_sh.at[pl.ds(row0, ROWS_PER_TILE)],
                    out_hbm.at[c, pl.ds(row0, ROWS_PER_TILE)])


R = 1000
GRID = N_NODES // R


def _mlp_stats_body(h_ref, agg_ref, w1_ref, w2_ref, p_ref, stats_ref):
    z = h_ref[...] + agg_ref[0] + agg_ref[1]
    y = jnp.maximum(jnp.dot(z, w1_ref[...], preferred_element_type=jnp.float32), 0.0)
    y = jnp.dot(y, w2_ref[...], preferred_element_type=jnp.float32)
    p = jnp.maximum(y, 0.0)
    p_ref[...] = p
    st = jnp.concatenate(
        [jnp.sum(p, axis=0, keepdims=True),
         jnp.sum(p * p, axis=0, keepdims=True)], axis=0)

    @pl.when(pl.program_id(0) == 0)
    def _():
        stats_ref[...] = jnp.zeros_like(stats_ref)

    stats_ref[...] += st


_mlp_stats = pl.pallas_call(
    _mlp_stats_body,
    grid=(GRID,),
    in_specs=[
        pl.BlockSpec((R, D), lambda i: (i, 0)),
        pl.BlockSpec((NC, R, D), lambda i: (0, i, 0)),
        pl.BlockSpec((D, D), lambda i: (0, 0)),
        pl.BlockSpec((D, D), lambda i: (0, 0)),
    ],
    out_specs=[
        pl.BlockSpec((R, D), lambda i: (i, 0)),
        pl.BlockSpec((2, D), lambda i: (0, 0)),
    ],
    out_shape=[
        jax.ShapeDtypeStruct((N_NODES, D), jnp.float32),
        jax.ShapeDtypeStruct((2, D), jnp.float32),
    ],
)


def _bn_body(p_ref, stats_ref, g_ref, b_ref, o_ref):
    inv_n = 1.0 / N_NODES
    mean = stats_ref[0] * inv_n
    var = stats_ref[1] * inv_n - mean * mean
    scale = g_ref[0] * lax.rsqrt(var + BN_EPS)
    shift = b_ref[0] - mean * scale
    o_ref[...] = p_ref[...] * scale + shift


_bn_apply = pl.pallas_call(
    _bn_body,
    grid=(GRID,),
    in_specs=[
        pl.BlockSpec((R, D), lambda i: (i, 0)),
        pl.BlockSpec((2, D), lambda i: (0, 0)),
        pl.BlockSpec((1, D), lambda i: (0, 0)),
        pl.BlockSpec((1, D), lambda i: (0, 0)),
    ],
    out_specs=pl.BlockSpec((R, D), lambda i: (i, 0)),
    out_shape=jax.ShapeDtypeStruct((N_NODES, D), jnp.float32),
)


def _final_body(h_ref, agg_ref, w1_ref, w2_ref, batch_ref, z_ref, pool_ref):
    z = h_ref[...] + agg_ref[0] + agg_ref[1]
    y = jnp.maximum(jnp.dot(z, w1_ref[...], preferred_element_type=jnp.float32), 0.0)
    y = jnp.dot(y, w2_ref[...], preferred_element_type=jnp.float32)
    z_ref[...] = y
    seg = lax.broadcasted_iota(jnp.int32, (N_GRAPHS, 1), 0)
    mask = (batch_ref[0] == seg).astype(jnp.float32)
    part = jnp.dot(mask, y, preferred_element_type=jnp.float32)

    @pl.when(pl.program_id(0) == 0)
    def _():
        pool_ref[...] = jnp.zeros_like(pool_ref)

    pool_ref[...] += part


_final_mlp_pool = pl.pallas_call(
    _final_body,
    grid=(GRID,),
    in_specs=[
        pl.BlockSpec((R, D), lambda i: (i, 0)),
        pl.BlockSpec((NC, R, D), lambda i: (0, i, 0)),
        pl.BlockSpec((D, D), lambda i: (0, 0)),
        pl.BlockSpec((D, D), lambda i: (0, 0)),
        pl.BlockSpec((1, 1, R), lambda i: (i, 0, 0)),
    ],
    out_specs=[
        pl.BlockSpec((R, D), lambda i: (i, 0)),
        pl.BlockSpec((N_GRAPHS, D), lambda i: (0, 0)),
    ],
    out_shape=[
        jax.ShapeDtypeStruct((N_NODES, D), jnp.float32),
        jax.ShapeDtypeStruct((N_GRAPHS, D), jnp.float32),
    ],
)


def kernel(x, edge_index, batch, W1_0, W2_0, g0, b0, W1_1, W2_1, g1, b1,
           W1_2, W2_2):
    src = edge_index[0].astype(jnp.int32)
    dst = edge_index[1].astype(jnp.int32)
    zeros = jnp.zeros((NPAD, D), jnp.float32)
    batch3d = batch.astype(jnp.int32).reshape(GRID, 1, R)
    g0r, b0r = g0.reshape(1, D), b0.reshape(1, D)
    g1r, b1r = g1.reshape(1, D), b1.reshape(1, D)

    h = x
    agg = _sc_segsum(h, src, dst, zeros)
    p, stats = _mlp_stats(h, agg, W1_0, W2_0)
    h1 = _bn_apply(p, stats, g0r, b0r)

    agg = _sc_segsum(h1, src, dst, zeros)
    p, stats = _mlp_stats(h1, agg, W1_1, W2_1)
    h2 = _bn_apply(p, stats, g1r, b1r)

    agg = _sc_segsum(h2, src, dst, zeros)
    z3, xpool = _final_mlp_pool(h2, agg, W1_2, W2_2, batch3d)

    return (xpool, jnp.concatenate([h1, h2, z3], axis=1))


# speedup vs baseline: 9.5015x; 1.9179x over previous
import functools

import jax
import jax.numpy as jnp
from jax import lax
from jax.experimental import pallas as pl
from jax.experimental.pallas import tpu as pltpu
from jax.experimental.pallas import tpu_sc as plsc

N_NODES = 10000
N_GRAPHS = 64
D = 128
E = 320000
BN_EPS = 1e-5

NC = 2
NS = 16
NW = NC * NS
EPW = E // NW
CH = 80
NCHUNK = EPW // CH
ROWS_PER_TILE = 632
NPAD = ROWS_PER_TILE * NS

_mesh = plsc.VectorSubcoreMesh(
    core_axis_name="c", subcore_axis_name="s", num_cores=NC, num_subcores=NS
)


NG = (NCHUNK - 1) // 2


@functools.partial(
    pl.kernel,
    mesh=_mesh,
    out_type=jax.ShapeDtypeStruct((NC, NPAD, D), jnp.float32),
    scratch_types=[
        pltpu.VMEM((CH,), jnp.int32),
        pltpu.VMEM((CH,), jnp.int32),
        pltpu.VMEM((CH,), jnp.int32),
        pltpu.VMEM((CH,), jnp.int32),
        pltpu.VMEM((CH, D), jnp.float32),
        pltpu.VMEM((CH, D), jnp.float32),
        pltpu.VMEM_SHARED((NPAD, D), jnp.float32),
        pltpu.SemaphoreType.DMA,
        pltpu.SemaphoreType.DMA,
        pltpu.SemaphoreType.DMA,
        pltpu.SemaphoreType.DMA,
    ],
)
def _sc_segsum(h_hbm, src_hbm, dst_hbm, zeros_hbm, out_hbm,
               src0, src1, dst0, dst1, rows0, rows1, agg_sh,
               isem0, isem1, gsem0, gsem1):
    c = lax.axis_index("c")
    s = lax.axis_index("s")
    wid = s * NC + c

    srcs = (src0, src1)
    dsts = (dst0, dst1)
    rows = (rows0, rows1)
    isems = (isem0, isem1)
    gsems = (gsem0, gsem1)

    row0 = pl.multiple_of(s * ROWS_PER_TILE, 8)
    pltpu.sync_copy(zeros_hbm.at[pl.ds(row0, ROWS_PER_TILE)],
                    agg_sh.at[pl.ds(row0, ROWS_PER_TILE)])
    plsc.subcore_barrier()

    base = wid * EPW

    def idx_start(i, b):
        off = pl.multiple_of(base + jnp.minimum(i, NCHUNK - 1) * CH, 8)
        pltpu.async_copy(src_hbm.at[pl.ds(off, CH)], srcs[b], isems[b])
        pltpu.async_copy(dst_hbm.at[pl.ds(off, CH)], dsts[b], isems[b])

    def idx_wait(b):
        pltpu.make_async_copy(src_hbm.at[pl.ds(0, CH)], srcs[b], isems[b]).wait()
        pltpu.make_async_copy(dst_hbm.at[pl.ds(0, CH)], dsts[b], isems[b]).wait()

    def gather_start(b):
        pltpu.async_copy(h_hbm.at[srcs[b]], rows[b], gsems[b])

    def gather_wait(b):
        pltpu.make_async_copy(h_hbm.at[srcs[b]], rows[b], gsems[b]).wait()

    def scatter(b):
        pltpu.sync_copy(rows[b], agg_sh.at[dsts[b]], add=True)

    idx_start(0, 0)
    idx_start(1, 1)
    idx_wait(0)
    gather_start(0)

    def outer(g, carry):
        i = 2 * g
        idx_wait(1)
        gather_start(1)
        gather_wait(0)
        scatter(0)
        idx_start(i + 2, 0)
        idx_wait(0)
        gather_start(0)
        gather_wait(1)
        scatter(1)
        idx_start(i + 3, 1)
        return carry

    lax.fori_loop(0, NG, outer, 0)
    gather_wait(0)
    scatter(0)
    idx_wait(1)
    plsc.subcore_barrier()

    pltpu.sync_copy(agg_sh.at[pl.ds(row0, ROWS_PER_TILE)],
                    out_hbm.at[c, pl.ds(row0, ROWS_PER_TILE)])


R = 1000
GRID = N_NODES // R


def _mlp_stats_body(h_ref, agg_ref, w1_ref, w2_ref, p_ref, stats_ref):
    z = h_ref[...] + agg_ref[0] + agg_ref[1]
    y = jnp.maximum(jnp.dot(z, w1_ref[...], preferred_element_type=jnp.float32), 0.0)
    y = jnp.dot(y, w2_ref[...], preferred_element_type=jnp.float32)
    p = jnp.maximum(y, 0.0)
    p_ref[...] = p
    st = jnp.concatenate(
        [jnp.sum(p, axis=0, keepdims=True),
         jnp.sum(p * p, axis=0, keepdims=True)], axis=0)

    @pl.when(pl.program_id(0) == 0)
    def _():
        stats_ref[...] = jnp.zeros_like(stats_ref)

    stats_ref[...] += st


_mlp_stats = pl.pallas_call(
    _mlp_stats_body,
    grid=(GRID,),
    in_specs=[
        pl.BlockSpec((R, D), lambda i: (i, 0)),
        pl.BlockSpec((NC, R, D), lambda i: (0, i, 0)),
        pl.BlockSpec((D, D), lambda i: (0, 0)),
        pl.BlockSpec((D, D), lambda i: (0, 0)),
    ],
    out_specs=[
        pl.BlockSpec((R, D), lambda i: (i, 0)),
        pl.BlockSpec((2, D), lambda i: (0, 0)),
    ],
    out_shape=[
        jax.ShapeDtypeStruct((N_NODES, D), jnp.float32),
        jax.ShapeDtypeStruct((2, D), jnp.float32),
    ],
)


def _bn_body(p_ref, stats_ref, g_ref, b_ref, o_ref):
    inv_n = 1.0 / N_NODES
    mean = stats_ref[0] * inv_n
    var = stats_ref[1] * inv_n - mean * mean
    scale = g_ref[0] * lax.rsqrt(var + BN_EPS)
    shift = b_ref[0] - mean * scale
    o_ref[...] = p_ref[...] * scale + shift


_bn_apply = pl.pallas_call(
    _bn_body,
    grid=(GRID,),
    in_specs=[
        pl.BlockSpec((R, D), lambda i: (i, 0)),
        pl.BlockSpec((2, D), lambda i: (0, 0)),
        pl.BlockSpec((1, D), lambda i: (0, 0)),
        pl.BlockSpec((1, D), lambda i: (0, 0)),
    ],
    out_specs=pl.BlockSpec((R, D), lambda i: (i, 0)),
    out_shape=jax.ShapeDtypeStruct((N_NODES, D), jnp.float32),
)


def _final_body(h_ref, agg_ref, w1_ref, w2_ref, batch_ref, z_ref, pool_ref):
    z = h_ref[...] + agg_ref[0] + agg_ref[1]
    y = jnp.maximum(jnp.dot(z, w1_ref[...], preferred_element_type=jnp.float32), 0.0)
    y = jnp.dot(y, w2_ref[...], preferred_element_type=jnp.float32)
    z_ref[...] = y
    seg = lax.broadcasted_iota(jnp.int32, (N_GRAPHS, 1), 0)
    mask = (batch_ref[0] == seg).astype(jnp.float32)
    part = jnp.dot(mask, y, preferred_element_type=jnp.float32)

    @pl.when(pl.program_id(0) == 0)
    def _():
        pool_ref[...] = jnp.zeros_like(pool_ref)

    pool_ref[...] += part


_final_mlp_pool = pl.pallas_call(
    _final_body,
    grid=(GRID,),
    in_specs=[
        pl.BlockSpec((R, D), lambda i: (i, 0)),
        pl.BlockSpec((NC, R, D), lambda i: (0, i, 0)),
        pl.BlockSpec((D, D), lambda i: (0, 0)),
        pl.BlockSpec((D, D), lambda i: (0, 0)),
        pl.BlockSpec((1, 1, R), lambda i: (i, 0, 0)),
    ],
    out_specs=[
        pl.BlockSpec((R, D), lambda i: (i, 0)),
        pl.BlockSpec((N_GRAPHS, D), lambda i: (0, 0)),
    ],
    out_shape=[
        jax.ShapeDtypeStruct((N_NODES, D), jnp.float32),
        jax.ShapeDtypeStruct((N_GRAPHS, D), jnp.float32),
    ],
)


def kernel(x, edge_index, batch, W1_0, W2_0, g0, b0, W1_1, W2_1, g1, b1,
           W1_2, W2_2):
    src = edge_index[0].astype(jnp.int32)
    dst = edge_index[1].astype(jnp.int32)
    zeros = jnp.zeros((NPAD, D), jnp.float32)
    batch3d = batch.astype(jnp.int32).reshape(GRID, 1, R)
    g0r, b0r = g0.reshape(1, D), b0.reshape(1, D)
    g1r, b1r = g1.reshape(1, D), b1.reshape(1, D)

    h = x
    agg = _sc_segsum(h, src, dst, zeros)
    p, stats = _mlp_stats(h, agg, W1_0, W2_0)
    h1 = _bn_apply(p, stats, g0r, b0r)

    agg = _sc_segsum(h1, src, dst, zeros)
    p, stats = _mlp_stats(h1, agg, W1_1, W2_1)
    h2 = _bn_apply(p, stats, g1r, b1r)

    agg = _sc_segsum(h2, src, dst, zeros)
    z3, xpool = _final_mlp_pool(h2, agg, W1_2, W2_2, batch3d)

    return (xpool, jnp.concatenate([h1, h2, z3], axis=1))


# speedup vs baseline: 10.8797x; 1.1451x over previous
import functools

import jax
import jax.numpy as jnp
from jax import lax
from jax.experimental import pallas as pl
from jax.experimental.pallas import tpu as pltpu
from jax.experimental.pallas import tpu_sc as plsc

N_NODES = 10000
N_GRAPHS = 64
D = 128
E = 320000
BN_EPS = 1e-5

NC = 2
NS = 16
NW = NC * NS
EPW = E // NW
CH = 80
NCHUNK = EPW // CH
ROWS_PER_TILE = 632
NPAD = ROWS_PER_TILE * NS

_mesh = plsc.VectorSubcoreMesh(
    core_axis_name="c", subcore_axis_name="s", num_cores=NC, num_subcores=NS
)


NB = 4
NG = (NCHUNK - 2 - 3) // NB


@functools.partial(
    pl.kernel,
    mesh=_mesh,
    out_type=jax.ShapeDtypeStruct((NC, NPAD, D), jnp.float32),
    scratch_types=(
        [pltpu.VMEM((CH,), jnp.int32) for _ in range(NB)]
        + [pltpu.VMEM((CH,), jnp.int32) for _ in range(NB)]
        + [pltpu.VMEM((CH, D), jnp.float32) for _ in range(NB)]
        + [pltpu.VMEM_SHARED((NPAD, D), jnp.float32)]
        + [pltpu.SemaphoreType.DMA for _ in range(3 * NB)]
    ),
)
def _sc_segsum(h_hbm, src_hbm, dst_hbm, zeros_hbm, out_hbm, *refs):
    srcs = refs[0:NB]
    dsts = refs[NB:2 * NB]
    rows = refs[2 * NB:3 * NB]
    agg_sh = refs[3 * NB]
    isems = refs[3 * NB + 1:3 * NB + 1 + NB]
    gsems = refs[3 * NB + 1 + NB:3 * NB + 1 + 2 * NB]
    ssems = refs[3 * NB + 1 + 2 * NB:3 * NB + 1 + 3 * NB]

    c = lax.axis_index("c")
    s = lax.axis_index("s")
    wid = s * NC + c

    row0 = pl.multiple_of(s * ROWS_PER_TILE, 8)
    pltpu.sync_copy(zeros_hbm.at[pl.ds(row0, ROWS_PER_TILE)],
                    agg_sh.at[pl.ds(row0, ROWS_PER_TILE)])
    plsc.subcore_barrier()

    base = wid * EPW

    def idx_start(i, b):
        off = pl.multiple_of(base + i * CH, 8)
        pltpu.async_copy(src_hbm.at[pl.ds(off, CH)], srcs[b], isems[b])
        pltpu.async_copy(dst_hbm.at[pl.ds(off, CH)], dsts[b], isems[b])

    def idx_wait(b):
        pltpu.make_async_copy(src_hbm.at[pl.ds(0, CH)], srcs[b], isems[b]).wait()
        pltpu.make_async_copy(dst_hbm.at[pl.ds(0, CH)], dsts[b], isems[b]).wait()

    def gather_start(b):
        pltpu.async_copy(h_hbm.at[srcs[b]], rows[b], gsems[b])

    def gather_wait(b):
        pltpu.make_async_copy(h_hbm.at[srcs[b]], rows[b], gsems[b]).wait()

    def scatter_start(b):
        pltpu.async_copy(rows[b], agg_sh.at[dsts[b]], ssems[b], add=True)

    def scatter_wait(b):
        pltpu.make_async_copy(rows[b], agg_sh.at[dsts[b]], ssems[b]).wait()

    for b in range(NB):
        idx_start(b, b)
    idx_wait(0)
    gather_start(0)
    idx_wait(1)
    gather_start(1)
    gather_wait(0)
    scatter_start(0)
    idx_wait(2)
    gather_start(2)
    gather_wait(1)
    scatter_start(1)

    def body(i, b):
        bn = (b + 1) % NB
        bp = (b + 2) % NB
        idx_wait(bn)
        gather_start(bn)
        gather_wait(b)
        scatter_start(b)
        scatter_wait(bp)
        idx_start(i + 2, bp)

    def outer(g, carry):
        i0 = 2 + NB * g
        for k in range(NB):
            body(i0 + k, (2 + k) % NB)
        return carry

    lax.fori_loop(0, NG, outer, 0)

    i = NCHUNK - 3
    idx_wait(3)
    gather_start(3)
    gather_wait(2)
    scatter_start(2)
    scatter_wait(0)
    idx_start(i + 2, 0)
    idx_wait(0)
    gather_start(0)
    gather_wait(3)
    scatter_start(3)
    scatter_wait(1)
    gather_wait(0)
    scatter_start(0)
    scatter_wait(2)
    scatter_wait(3)
    scatter_wait(0)

    plsc.subcore_barrier()

    pltpu.sync_copy(agg_sh.at[pl.ds(row0, ROWS_PER_TILE)],
                    out_hbm.at[c, pl.ds(row0, ROWS_PER_TILE)])


R = 1000
GRID = N_NODES // R


def _mlp_stats_body(h_ref, agg_ref, w1_ref, w2_ref, p_ref, stats_ref):
    z = h_ref[...] + agg_ref[0] + agg_ref[1]
    y = jnp.maximum(jnp.dot(z, w1_ref[...], preferred_element_type=jnp.float32), 0.0)
    y = jnp.dot(y, w2_ref[...], preferred_element_type=jnp.float32)
    p = jnp.maximum(y, 0.0)
    p_ref[...] = p
    st = jnp.concatenate(
        [jnp.sum(p, axis=0, keepdims=True),
         jnp.sum(p * p, axis=0, keepdims=True)], axis=0)

    @pl.when(pl.program_id(0) == 0)
    def _():
        stats_ref[...] = jnp.zeros_like(stats_ref)

    stats_ref[...] += st


_mlp_stats = pl.pallas_call(
    _mlp_stats_body,
    grid=(GRID,),
    in_specs=[
        pl.BlockSpec((R, D), lambda i: (i, 0)),
        pl.BlockSpec((NC, R, D), lambda i: (0, i, 0)),
        pl.BlockSpec((D, D), lambda i: (0, 0)),
        pl.BlockSpec((D, D), lambda i: (0, 0)),
    ],
    out_specs=[
        pl.BlockSpec((R, D), lambda i: (i, 0)),
        pl.BlockSpec((2, D), lambda i: (0, 0)),
    ],
    out_shape=[
        jax.ShapeDtypeStruct((N_NODES, D), jnp.float32),
        jax.ShapeDtypeStruct((2, D), jnp.float32),
    ],
)


def _bn_body(p_ref, stats_ref, g_ref, b_ref, o_ref):
    inv_n = 1.0 / N_NODES
    mean = stats_ref[0] * inv_n
    var = stats_ref[1] * inv_n - mean * mean
    scale = g_ref[0] * lax.rsqrt(var + BN_EPS)
    shift = b_ref[0] - mean * scale
    o_ref[...] = p_ref[...] * scale + shift


_bn_apply = pl.pallas_call(
    _bn_body,
    grid=(GRID,),
    in_specs=[
        pl.BlockSpec((R, D), lambda i: (i, 0)),
        pl.BlockSpec((2, D), lambda i: (0, 0)),
        pl.BlockSpec((1, D), lambda i: (0, 0)),
        pl.BlockSpec((1, D), lambda i: (0, 0)),
    ],
    out_specs=pl.BlockSpec((R, D), lambda i: (i, 0)),
    out_shape=jax.ShapeDtypeStruct((N_NODES, D), jnp.float32),
)


def _final_body(h_ref, agg_ref, w1_ref, w2_ref, batch_ref, z_ref, pool_ref):
    z = h_ref[...] + agg_ref[0] + agg_ref[1]
    y = jnp.maximum(jnp.dot(z, w1_ref[...], preferred_element_type=jnp.float32), 0.0)
    y = jnp.dot(y, w2_ref[...], preferred_element_type=jnp.float32)
    z_ref[...] = y
    seg = lax.broadcasted_iota(jnp.int32, (N_GRAPHS, 1), 0)
    mask = (batch_ref[0] == seg).astype(jnp.float32)
    part = jnp.dot(mask, y, preferred_element_type=jnp.float32)

    @pl.when(pl.program_id(0) == 0)
    def _():
        pool_ref[...] = jnp.zeros_like(pool_ref)

    pool_ref[...] += part


_final_mlp_pool = pl.pallas_call(
    _final_body,
    grid=(GRID,),
    in_specs=[
        pl.BlockSpec((R, D), lambda i: (i, 0)),
        pl.BlockSpec((NC, R, D), lambda i: (0, i, 0)),
        pl.BlockSpec((D, D), lambda i: (0, 0)),
        pl.BlockSpec((D, D), lambda i: (0, 0)),
        pl.BlockSpec((1, 1, R), lambda i: (i, 0, 0)),
    ],
    out_specs=[
        pl.BlockSpec((R, D), lambda i: (i, 0)),
        pl.BlockSpec((N_GRAPHS, D), lambda i: (0, 0)),
    ],
    out_shape=[
        jax.ShapeDtypeStruct((N_NODES, D), jnp.float32),
        jax.ShapeDtypeStruct((N_GRAPHS, D), jnp.float32),
    ],
)


def kernel(x, edge_index, batch, W1_0, W2_0, g0, b0, W1_1, W2_1, g1, b1,
           W1_2, W2_2):
    src = edge_index[0].astype(jnp.int32)
    dst = edge_index[1].astype(jnp.int32)
    zeros = jnp.zeros((NPAD, D), jnp.float32)
    batch3d = batch.astype(jnp.int32).reshape(GRID, 1, R)
    g0r, b0r = g0.reshape(1, D), b0.reshape(1, D)
    g1r, b1r = g1.reshape(1, D), b1.reshape(1, D)

    h = x
    agg = _sc_segsum(h, src, dst, zeros)
    p, stats = _mlp_stats(h, agg, W1_0, W2_0)
    h1 = _bn_apply(p, stats, g0r, b0r)

    agg = _sc_segsum(h1, src, dst, zeros)
    p, stats = _mlp_stats(h1, agg, W1_1, W2_1)
    h2 = _bn_apply(p, stats, g1r, b1r)

    agg = _sc_segsum(h2, src, dst, zeros)
    z3, xpool = _final_mlp_pool(h2, agg, W1_2, W2_2, batch3d)

    return (xpool, jnp.concatenate([h1, h2, z3], axis=1))


# speedup vs baseline: 13.1118x; 1.2052x over previous
import functools

import jax
import jax.numpy as jnp
from jax import lax
from jax.experimental import pallas as pl
from jax.experimental.pallas import tpu as pltpu
from jax.experimental.pallas import tpu_sc as plsc

N_NODES = 10000
N_GRAPHS = 64
D = 128
E = 320000
BN_EPS = 1e-5

NC = 2
NS = 16
NW = NC * NS
EPW = E // NW
CH = 96
NCHUNK = EPW // CH
TAIL = EPW - NCHUNK * CH
ROWS_PER_TILE = 632
NPAD = ROWS_PER_TILE * NS

_mesh = plsc.VectorSubcoreMesh(
    core_axis_name="c", subcore_axis_name="s", num_cores=NC, num_subcores=NS
)


NBR = 3
NBI = 6
GD = 2
ID = 5
SW = NBR - GD
NSTEADY = (NCHUNK - 1 - ID - 2 + 1) // NBI
STEADY_END = 2 + NBI * NSTEADY


@functools.partial(
    pl.kernel,
    mesh=_mesh,
    out_type=jax.ShapeDtypeStruct((NC, NPAD, D), jnp.float32),
    scratch_types=(
        [pltpu.VMEM((CH,), jnp.int32) for _ in range(2 * NBI)]
        + [pltpu.VMEM((CH, D), jnp.float32) for _ in range(NBR)]
        + [pltpu.VMEM((TAIL,), jnp.int32) for _ in range(2)]
        + [pltpu.VMEM((TAIL, D), jnp.float32)]
        + [pltpu.VMEM_SHARED((NPAD, D), jnp.float32)]
        + [pltpu.SemaphoreType.DMA for _ in range(NBI + 2 * NBR + 1)]
    ),
)
def _sc_segsum(h_hbm, src_hbm, dst_hbm, zeros_hbm, out_hbm, *refs):
    srcs = refs[0:NBI]
    dsts = refs[NBI:2 * NBI]
    rows = refs[2 * NBI:2 * NBI + NBR]
    src_t, dst_t, rows_t = refs[2 * NBI + NBR:2 * NBI + NBR + 3]
    agg_sh = refs[2 * NBI + NBR + 3]
    sems = refs[2 * NBI + NBR + 4:]
    isems = sems[0:NBI]
    gsems = sems[NBI:NBI + NBR]
    ssems = sems[NBI + NBR:NBI + 2 * NBR]
    tsem = sems[NBI + 2 * NBR]

    c = lax.axis_index("c")
    s = lax.axis_index("s")
    wid = s * NC + c

    row0 = pl.multiple_of(s * ROWS_PER_TILE, 8)
    pltpu.sync_copy(zeros_hbm.at[pl.ds(row0, ROWS_PER_TILE)],
                    agg_sh.at[pl.ds(row0, ROWS_PER_TILE)])
    plsc.subcore_barrier()

    base = wid * EPW

    def idx_start(i, b):
        off = pl.multiple_of(base + i * CH, 8)
        pltpu.async_copy(src_hbm.at[pl.ds(off, CH)], srcs[b], isems[b])
        pltpu.async_copy(dst_hbm.at[pl.ds(off, CH)], dsts[b], isems[b])

    def idx_wait(b):
        pltpu.make_async_copy(src_hbm.at[pl.ds(0, CH)], srcs[b], isems[b]).wait()
        pltpu.make_async_copy(dst_hbm.at[pl.ds(0, CH)], dsts[b], isems[b]).wait()

    def gather_start(bi, br):
        pltpu.async_copy(h_hbm.at[srcs[bi]], rows[br], gsems[br])

    def gather_wait(bi, br):
        pltpu.make_async_copy(h_hbm.at[srcs[bi]], rows[br], gsems[br]).wait()

    def scatter_start(bi, br):
        pltpu.async_copy(rows[br], agg_sh.at[dsts[bi]], ssems[br], add=True)

    def scatter_wait(bi, br):
        pltpu.make_async_copy(rows[br], agg_sh.at[dsts[bi]], ssems[br]).wait()

    def emit_body(i, r, first=False, last_g=True, last_i=True):
        if last_g:
            idx_wait((r + GD) % NBI)
        if not first:
            scatter_wait((r + GD) % NBI, (r + GD) % NBR)
        if last_g:
            gather_start((r + GD) % NBI, (r + GD) % NBR)
        gather_wait(r, r % NBR)
        scatter_start(r, r % NBR)
        if last_i:
            idx_start(i + ID, (r + ID) % NBI)

    for j in range(ID):
        idx_start(j, j)
    for j in range(GD):
        idx_wait(j)
        gather_start(j, j)
    for j in range(2):
        emit_body(j, j, first=(j < SW))

    def outer(g, carry):
        i0 = 2 + NBI * g
        for k in range(NBI):
            emit_body(i0 + k, (2 + k) % NBI)
        return carry

    lax.fori_loop(0, NSTEADY, outer, 0)

    for i in range(STEADY_END, NCHUNK):
        emit_body(i, i % NBI, last_g=(i + GD < NCHUNK), last_i=(i + ID < NCHUNK))
    for i in range(NCHUNK - SW, NCHUNK):
        scatter_wait(i % NBI, i % NBR)

    toff = pl.multiple_of(base + NCHUNK * CH, 8)
    pltpu.sync_copy(src_hbm.at[pl.ds(toff, TAIL)], src_t)
    pltpu.sync_copy(dst_hbm.at[pl.ds(toff, TAIL)], dst_t)
    pltpu.async_copy(h_hbm.at[src_t], rows_t, tsem).wait()
    pltpu.sync_copy(rows_t, agg_sh.at[dst_t], add=True)

    plsc.subcore_barrier()

    pltpu.sync_copy(agg_sh.at[pl.ds(row0, ROWS_PER_TILE)],
                    out_hbm.at[c, pl.ds(row0, ROWS_PER_TILE)])


R = 1000
GRID = N_NODES // R


def _mlp_stats_body(h_ref, agg_ref, w1_ref, w2_ref, p_ref, stats_ref):
    z = h_ref[...] + agg_ref[0] + agg_ref[1]
    y = jnp.maximum(jnp.dot(z, w1_ref[...], preferred_element_type=jnp.float32), 0.0)
    y = jnp.dot(y, w2_ref[...], preferred_element_type=jnp.float32)
    p = jnp.maximum(y, 0.0)
    p_ref[...] = p
    st = jnp.concatenate(
        [jnp.sum(p, axis=0, keepdims=True),
         jnp.sum(p * p, axis=0, keepdims=True)], axis=0)

    @pl.when(pl.program_id(0) == 0)
    def _():
        stats_ref[...] = jnp.zeros_like(stats_ref)

    stats_ref[...] += st


_mlp_stats = pl.pallas_call(
    _mlp_stats_body,
    grid=(GRID,),
    in_specs=[
        pl.BlockSpec((R, D), lambda i: (i, 0)),
        pl.BlockSpec((NC, R, D), lambda i: (0, i, 0)),
        pl.BlockSpec((D, D), lambda i: (0, 0)),
        pl.BlockSpec((D, D), lambda i: (0, 0)),
    ],
    out_specs=[
        pl.BlockSpec((R, D), lambda i: (i, 0)),
        pl.BlockSpec((2, D), lambda i: (0, 0)),
    ],
    out_shape=[
        jax.ShapeDtypeStruct((N_NODES, D), jnp.float32),
        jax.ShapeDtypeStruct((2, D), jnp.float32),
    ],
)


def _bn_body(p_ref, stats_ref, g_ref, b_ref, o_ref):
    inv_n = 1.0 / N_NODES
    mean = stats_ref[0] * inv_n
    var = stats_ref[1] * inv_n - mean * mean
    scale = g_ref[0] * lax.rsqrt(var + BN_EPS)
    shift = b_ref[0] - mean * scale
    o_ref[...] = p_ref[...] * scale + shift


_bn_apply = pl.pallas_call(
    _bn_body,
    grid=(GRID,),
    in_specs=[
        pl.BlockSpec((R, D), lambda i: (i, 0)),
        pl.BlockSpec((2, D), lambda i: (0, 0)),
        pl.BlockSpec((1, D), lambda i: (0, 0)),
        pl.BlockSpec((1, D), lambda i: (0, 0)),
    ],
    out_specs=pl.BlockSpec((R, D), lambda i: (i, 0)),
    out_shape=jax.ShapeDtypeStruct((N_NODES, D), jnp.float32),
)


def _final_body(h_ref, agg_ref, w1_ref, w2_ref, batch_ref, z_ref, pool_ref):
    z = h_ref[...] + agg_ref[0] + agg_ref[1]
    y = jnp.maximum(jnp.dot(z, w1_ref[...], preferred_element_type=jnp.float32), 0.0)
    y = jnp.dot(y, w2_ref[...], preferred_element_type=jnp.float32)
    z_ref[...] = y
    seg = lax.broadcasted_iota(jnp.int32, (N_GRAPHS, 1), 0)
    mask = (batch_ref[0] == seg).astype(jnp.float32)
    part = jnp.dot(mask, y, preferred_element_type=jnp.float32)

    @pl.when(pl.program_id(0) == 0)
    def _():
        pool_ref[...] = jnp.zeros_like(pool_ref)

    pool_ref[...] += part


_final_mlp_pool = pl.pallas_call(
    _final_body,
    grid=(GRID,),
    in_specs=[
        pl.BlockSpec((R, D), lambda i: (i, 0)),
        pl.BlockSpec((NC, R, D), lambda i: (0, i, 0)),
        pl.BlockSpec((D, D), lambda i: (0, 0)),
        pl.BlockSpec((D, D), lambda i: (0, 0)),
        pl.BlockSpec((1, 1, R), lambda i: (i, 0, 0)),
    ],
    out_specs=[
        pl.BlockSpec((R, D), lambda i: (i, 0)),
        pl.BlockSpec((N_GRAPHS, D), lambda i: (0, 0)),
    ],
    out_shape=[
        jax.ShapeDtypeStruct((N_NODES, D), jnp.float32),
        jax.ShapeDtypeStruct((N_GRAPHS, D), jnp.float32),
    ],
)


def kernel(x, edge_index, batch, W1_0, W2_0, g0, b0, W1_1, W2_1, g1, b1,
           W1_2, W2_2):
    src = edge_index[0].astype(jnp.int32)
    dst = edge_index[1].astype(jnp.int32)
    zeros = jnp.zeros((NPAD, D), jnp.float32)
    batch3d = batch.astype(jnp.int32).reshape(GRID, 1, R)
    g0r, b0r = g0.reshape(1, D), b0.reshape(1, D)
    g1r, b1r = g1.reshape(1, D), b1.reshape(1, D)

    h = x
    agg = _sc_segsum(h, src, dst, zeros)
    p, stats = _mlp_stats(h, agg, W1_0, W2_0)
    h1 = _bn_apply(p, stats, g0r, b0r)

    agg = _sc_segsum(h1, src, dst, zeros)
    p, stats = _mlp_stats(h1, agg, W1_1, W2_1)
    h2 = _bn_apply(p, stats, g1r, b1r)

    agg = _sc_segsum(h2, src, dst, zeros)
    z3, xpool = _final_mlp_pool(h2, agg, W1_2, W2_2, batch3d)

    return (xpool, jnp.concatenate([h1, h2, z3], axis=1))
